# baseline (speedup 1.0000x reference)
import functools

import jax
import jax.numpy as jnp
from jax.experimental import pallas as pl
from jax.experimental.pallas import tpu as pltpu

_C = 512
_B = 8
_L = 1024
_NTOK = _B * _L
_STEPS = 3
_CHUNK = 512
_NCHUNK = _NTOK // _CHUNK
_GCHUNK = 512
_PREC = jax.lax.Precision.HIGHEST
_PREC_MM = jax.lax.Precision.DEFAULT
_NEG = -1e30


def _lstm_act(gates, c):
    ig = jax.nn.sigmoid(gates[:, 0 * _C:1 * _C])
    fg = jax.nn.sigmoid(gates[:, 1 * _C:2 * _C])
    gg = jnp.tanh(gates[:, 2 * _C:3 * _C])
    og = jax.nn.sigmoid(gates[:, 3 * _C:4 * _C])
    c = fg * c + ig * gg
    return og * jnp.tanh(c), c


def _fused_body(x_ref, batch_ref, qstar_ref, w_ref, wih_ref, whh_ref,
                wc_ref, wr_ref, b_ref, out_ref, h_ref, g_ref):
    def asit_step(i, _):
        wrow = w_ref[pl.ds(i, 1), :]
        segx = x_ref[pl.ds(i * _L, _L), :]
        h_ref[pl.ds(i, 1), :] = jax.lax.dot_general(
            wrow, segx, (((1,), (0,)), ((), ())), precision=_PREC_MM)
        return 0

    jax.lax.fori_loop(0, _B, asit_step, 0)
    h = h_ref[...]
    c = jnp.zeros((_B, _C), jnp.float32)
    bias = b_ref[...]
    iota_b = jax.lax.broadcasted_iota(jnp.int32, (_B, _CHUNK), 0)

    def gates_of(lhs1, w1_ref, lhs2, w2_ref):
        def gate_chunk(g, _):
            w1c = w1_ref[pl.ds(g * _GCHUNK, _GCHUNK), :]
            w2c = w2_ref[pl.ds(g * _GCHUNK, _GCHUNK), :]
            g_ref[:, pl.ds(g * _GCHUNK, _GCHUNK)] = (
                jax.lax.dot_general(lhs1, w1c, (((1,), (1,)), ((), ())),
                                    precision=_PREC_MM)
                + jax.lax.dot_general(lhs2, w2c, (((1,), (1,)), ((), ())),
                                      precision=_PREC_MM))
            return 0

        jax.lax.fori_loop(0, (4 * _C) // _GCHUNK, gate_chunk, 0)
        return g_ref[...]

    def pool(h):
        def chunk(j, carry):
            m, denom, racc = carry
            xc = x_ref[pl.ds(j * _CHUNK, _CHUNK), :]
            segc = batch_ref[:, pl.ds(j * _CHUNK, _CHUNK)]
            oh = iota_b == segc
            s = jax.lax.dot_general(h, xc, (((1,), (1,)), ((), ())),
                                    precision=_PREC)
            smask = jnp.where(oh, s, _NEG)
            m_new = jnp.maximum(m, jnp.max(smask, axis=1, keepdims=True))
            scale = jnp.exp(m - m_new)
            p = jnp.exp(smask - m_new)
            denom = denom * scale + jnp.sum(p, axis=1, keepdims=True)
            racc = racc * scale + jax.lax.dot_general(
                p, xc, (((1,), (0,)), ((), ())), precision=_PREC)
            return m_new, denom, racc

        m0 = jnp.full((_B, 1), _NEG, jnp.float32)
        z1 = jnp.zeros((_B, 1), jnp.float32)
        z2 = jnp.zeros((_B, _C), jnp.float32)
        _, denom, racc = jax.lax.fori_loop(0, _NCHUNK, chunk, (m0, z1, z2))
        return racc / (denom + 1e-16)

    qs = qstar_ref[...]
    h, c = _lstm_act(gates_of(qs, wih_ref, h, whh_ref) + bias, c)
    r = pool(h)

    for _ in range(_STEPS - 1):
        h, c = _lstm_act(gates_of(h, wc_ref, r, wr_ref) + bias, c)
        r = pool(h)

    out_ref[...] = jnp.concatenate([h, r], axis=1)


@functools.partial(jax.jit, static_argnames=("interpret",))
def _run_fused(x, seg_row, q_star, w_rows, W_ih, W_hh, Wc, Wr, bias,
               interpret=False):
    return pl.pallas_call(
        _fused_body,
        out_shape=jax.ShapeDtypeStruct((_B, 2 * _C), jnp.float32),
        scratch_shapes=[
            pltpu.VMEM((_B, _C), jnp.float32),
            pltpu.VMEM((_B, 4 * _C), jnp.float32),
        ],
        interpret=interpret,
    )(x, seg_row, q_star, w_rows, W_ih, W_hh, Wc, Wr, bias)


def kernel(x, batch, q_star, bank_s_list, bank_s, index, cuda,
           W_ih, W_hh, b_ih, b_hh, interpret=False):
    w_rows = jax.lax.dynamic_slice_in_dim(
        bank_s_list, index, 1, axis=1).reshape(_B, _L)
    seg_row = batch.astype(jnp.int32).reshape(1, _NTOK)
    bias = jnp.broadcast_to((b_ih + b_hh).reshape(1, 4 * _C), (_B, 4 * _C))
    Wc = W_ih[:, :_C] + W_hh
    Wr = W_ih[:, _C:]
    return _run_fused(x, seg_row, q_star, w_rows, W_ih, W_hh, Wc, Wr, bias,
                      interpret=interpret)
